# Initial kernel scaffold; baseline (speedup 1.0000x reference)
import jax
import jax.numpy as jnp
from jax.experimental import pallas as pl


def kernel(input_ids, token_table, pos_table, gamma, beta):
    raise NotImplementedError("write your pallas kernel here")


# speedup vs baseline: 3.2788x; 3.2788x over previous
import functools

import jax
import jax.numpy as jnp
from jax import lax
from jax.experimental import pallas as pl
from jax.experimental.pallas import tpu as pltpu
from jax.experimental.pallas import tpu_sc as plsc

EPS = 1e-3

_NC = 2
_NS = 16
_NW = _NC * _NS


def _sc_gather(table, idx_flat, chunk=128):
    n = idx_flat.shape[0]
    d = table.shape[1]
    b_per_w = n // _NW
    n_chunks = b_per_w // chunk
    assert b_per_w % chunk == 0

    mesh = plsc.VectorSubcoreMesh(core_axis_name="c", subcore_axis_name="s")

    @functools.partial(
        pl.kernel,
        mesh=mesh,
        out_type=jax.ShapeDtypeStruct((n, d), jnp.float32),
        scratch_types=[
            pltpu.VMEM((chunk,), jnp.int32),
            pltpu.VMEM((chunk, d), jnp.float32),
            pltpu.SemaphoreType.DMA,
        ],
    )
    def k(table_hbm, idx_hbm, out_hbm, idx_v, rows_v, sem):
        wid = lax.axis_index("s") * _NC + lax.axis_index("c")

        @pl.loop(0, n_chunks)
        def _(i):
            base = wid * b_per_w + i * chunk
            pltpu.sync_copy(idx_hbm.at[pl.ds(base, chunk)], idx_v)
            pltpu.async_copy(table_hbm.at[idx_v], rows_v, sem).wait()
            pltpu.sync_copy(rows_v, out_hbm.at[pl.ds(base, chunk)])

    return k(table, idx_flat)


def _ln_body(x_ref, pos_ref, g_ref, b_ref, o_ref):
    x = x_ref[...] + pos_ref[...]
    mean = jnp.mean(x, axis=-1, keepdims=True)
    var = jnp.mean(jnp.square(x - mean), axis=-1, keepdims=True)
    inv = lax.rsqrt(var + EPS)
    o_ref[...] = (x - mean) * inv * g_ref[...][0] + b_ref[...][0]


def _tc_ln(gathered, pos_table, gamma, beta, seq_block=8):
    b, s, h = gathered.shape
    grid = (b // seq_block,)
    return pl.pallas_call(
        _ln_body,
        grid=grid,
        in_specs=[
            pl.BlockSpec((seq_block, s, h), lambda i: (i, 0, 0)),
            pl.BlockSpec((1, s, h), lambda i: (0, 0, 0)),
            pl.BlockSpec((1, h), lambda i: (0, 0)),
            pl.BlockSpec((1, h), lambda i: (0, 0)),
        ],
        out_specs=pl.BlockSpec((seq_block, s, h), lambda i: (i, 0, 0)),
        out_shape=jax.ShapeDtypeStruct((b, s, h), jnp.float32),
    )(gathered, pos_table[None, :, :], gamma[None, :], beta[None, :])


def kernel(input_ids, token_table, pos_table, gamma, beta):
    b, s = input_ids.shape
    h = token_table.shape[1]
    gathered = _sc_gather(token_table, input_ids.reshape(-1))
    return _tc_ln(gathered.reshape(b, s, h), pos_table, gamma, beta)


# speedup vs baseline: 4.0645x; 1.2396x over previous
import functools

import jax
import jax.numpy as jnp
from jax import lax
from jax.experimental import pallas as pl
from jax.experimental.pallas import tpu as pltpu
from jax.experimental.pallas import tpu_sc as plsc

EPS = 1e-3

_NC = 2
_NS = 16
_NW = _NC * _NS


def _sc_gather(table, idx_flat, chunk=128, nbuf=5):
    n = idx_flat.shape[0]
    d = table.shape[1]
    b_per_w = n // _NW
    n_chunks = b_per_w // chunk
    assert b_per_w % chunk == 0 and n_chunks % nbuf == 0

    mesh = plsc.VectorSubcoreMesh(core_axis_name="c", subcore_axis_name="s")

    @functools.partial(
        pl.kernel,
        mesh=mesh,
        out_type=jax.ShapeDtypeStruct((n, d), jnp.float32),
        scratch_types=[
            pltpu.VMEM((b_per_w,), jnp.int32),
            pltpu.VMEM((nbuf, chunk, d), jnp.float32),
        ]
        + [pltpu.SemaphoreType.DMA] * (2 * nbuf),
    )
    def k(table_hbm, idx_hbm, out_hbm, idx_v, rows_v, *sems):
        sg, sw = sems[:nbuf], sems[nbuf:]
        wid = lax.axis_index("s") * _NC + lax.axis_index("c")
        base_w = wid * b_per_w
        pltpu.sync_copy(idx_hbm.at[pl.ds(base_w, b_per_w)], idx_v)

        @pl.loop(0, n_chunks, step=nbuf)
        def _(i):
            gathers = []
            for b in range(nbuf):
                idx_slice = idx_v.at[pl.ds((i + b) * chunk, chunk)]
                gathers.append(
                    pltpu.async_copy(table_hbm.at[idx_slice], rows_v.at[b], sg[b])
                )
            writes = []
            for b in range(nbuf):
                gathers[b].wait()
                writes.append(
                    pltpu.async_copy(
                        rows_v.at[b],
                        out_hbm.at[pl.ds(base_w + (i + b) * chunk, chunk)],
                        sw[b],
                    )
                )
            for w in writes:
                w.wait()

    return k(table, idx_flat)


def _ln_body(x_ref, pos_ref, g_ref, b_ref, o_ref):
    x = x_ref[...] + pos_ref[...]
    mean = jnp.mean(x, axis=-1, keepdims=True)
    var = jnp.mean(jnp.square(x - mean), axis=-1, keepdims=True)
    inv = lax.rsqrt(var + EPS)
    o_ref[...] = (x - mean) * inv * g_ref[...][0] + b_ref[...][0]


def _tc_ln(gathered, pos_table, gamma, beta, seq_block=8):
    b, s, h = gathered.shape
    grid = (b // seq_block,)
    return pl.pallas_call(
        _ln_body,
        grid=grid,
        in_specs=[
            pl.BlockSpec((seq_block, s, h), lambda i: (i, 0, 0)),
            pl.BlockSpec((1, s, h), lambda i: (0, 0, 0)),
            pl.BlockSpec((1, h), lambda i: (0, 0)),
            pl.BlockSpec((1, h), lambda i: (0, 0)),
        ],
        out_specs=pl.BlockSpec((seq_block, s, h), lambda i: (i, 0, 0)),
        out_shape=jax.ShapeDtypeStruct((b, s, h), jnp.float32),
    )(gathered, pos_table[None, :, :], gamma[None, :], beta[None, :])


def kernel(input_ids, token_table, pos_table, gamma, beta):
    b, s = input_ids.shape
    h = token_table.shape[1]
    gathered = _sc_gather(token_table, input_ids.reshape(-1))
    return _tc_ln(gathered.reshape(b, s, h), pos_table, gamma, beta)
